# baseline (speedup 1.0000x reference)
import jax
import jax.numpy as jnp
from jax.experimental import pallas as pl
from jax.experimental.pallas import tpu as pltpu

N = 4096
D = 256
BM = 256
NB = N // BM


def _fused_gcn_kernel(adj_ref, x_ref, w1_ref, b1_ref, w2_ref, b2_ref,
                      o_ref, adjbf_ref, s2_ref):
    m = pl.program_id(0)

    for c in range(2, NB + 2):
        @pl.when(m == c)
        def _(c=c):
            b = c - 2
            r0, r1 = b * BM, (b + 1) * BM
            s2_b = s2_ref[r0:r1, :]
            diag = jnp.dot(
                adjbf_ref[r0:r1, r0:r1], s2_b,
                preferred_element_type=jnp.float32,
            )
            if b < NB - 1:
                o_ref[r0:r1, :] = (
                    jnp.broadcast_to(b2_ref[...], (BM, D)) + diag
                )
            else:
                o_ref[r0:r1, :] += diag
            if b > 0:
                o_ref[:r0, :] += jnp.dot(
                    adjbf_ref[:r0, r0:r1], s2_b,
                    preferred_element_type=jnp.float32,
                )

    @pl.when(m == NB)
    def _():
        o_ref[N - BM:, :] = jnp.broadcast_to(b2_ref[...], (BM, D))

    for c in (NB, NB + 1):
        @pl.when(m == c)
        def _(c=c):
            for k in range(c - NB, NB - 1, 2):
                r0, r1 = k * BM, (k + 1) * BM
                o_ref[r1:, :] += jnp.dot(
                    adjbf_ref[r1:, r0:r1], s2_ref[r0:r1, :],
                    preferred_element_type=jnp.float32,
                )

    @pl.when(jnp.logical_and(m >= 1, m <= NB))
    def _():
        b0 = (m - 1) * BM
        arow = adjbf_ref[pl.ds(b0, BM), :]
        u = jnp.dot(arow, x_ref[...], preferred_element_type=jnp.float32)
        t = jnp.dot(
            u.astype(jnp.bfloat16), w1_ref[...],
            preferred_element_type=jnp.float32,
        )
        h = jnp.maximum(t + b1_ref[...], 0.0).astype(jnp.bfloat16)
        s2_ref[pl.ds(b0, BM), :] = jnp.dot(
            h, w2_ref[...], preferred_element_type=jnp.float32
        ).astype(jnp.bfloat16)

    @pl.when(m < NB)
    def _():
        adjbf_ref[pl.ds(m * BM, BM), :] = adj_ref[...].astype(jnp.bfloat16)


def kernel(x, adj, W1, b1, W2, b2):
    xb = x.astype(jnp.bfloat16)
    w1b = W1.astype(jnp.bfloat16)
    w2b = W2.astype(jnp.bfloat16)
    b1r = b1.reshape(1, D)
    b2r = b2.reshape(1, D)
    return pl.pallas_call(
        _fused_gcn_kernel,
        grid=(NB + 2,),
        in_specs=[
            pl.BlockSpec((BM, N), lambda i: (jnp.minimum(i, NB - 1), 0)),
            pl.BlockSpec((N, D), lambda i: (0, 0)),
            pl.BlockSpec((D, D), lambda i: (0, 0)),
            pl.BlockSpec((1, D), lambda i: (0, 0)),
            pl.BlockSpec((D, D), lambda i: (0, 0)),
            pl.BlockSpec((1, D), lambda i: (0, 0)),
        ],
        out_specs=pl.BlockSpec((N, D), lambda i: (0, 0)),
        out_shape=jax.ShapeDtypeStruct((N, D), jnp.float32),
        scratch_shapes=[
            pltpu.VMEM((N, N), jnp.bfloat16),
            pltpu.VMEM((N, D), jnp.bfloat16),
        ],
    )(adj, xb, w1b, b1r, w2b, b2r)


# speedup vs baseline: 1.0248x; 1.0248x over previous
import jax
import jax.numpy as jnp
from jax.experimental import pallas as pl
from jax.experimental.pallas import tpu as pltpu

N = 4096
D = 256
SB = 512
NSB = N // SB
HW = N // 2
CHUNK = 2048


def _fused_gcn_kernel(adj_ref, x_ref, w1_ref, b1_ref, w2_ref, b2_ref,
                      o_ref, adjbf_ref, s2_ref):
    m = pl.program_id(0)

    for c in range(3, 2 * NSB + 2, 2):
        @pl.when(m == c)
        def _(c=c):
            b = (c - 3) // 2
            r0, r1 = b * SB, (b + 1) * SB
            s2_b = s2_ref[r0:r1, :]
            o_ref[r0:r1, :] = jnp.broadcast_to(b2_ref[...], (SB, D)) + jnp.dot(
                adjbf_ref[r0:r1, :r1], s2_ref[:r1, :],
                preferred_element_type=jnp.float32,
            )
            for q0 in range(0, r0, CHUNK):
                q1 = min(q0 + CHUNK, r0)
                o_ref[q0:q1, :] += jnp.dot(
                    adjbf_ref[q0:q1, r0:r1], s2_b,
                    preferred_element_type=jnp.float32,
                )

    @pl.when(jnp.logical_and(m >= 2, jnp.logical_and(m <= 2 * NSB,
                                                     m % 2 == 0)))
    def _():
        b0 = (m - 2) // 2 * SB
        arow = adjbf_ref[pl.ds(b0, SB), :]
        u = jnp.dot(arow, x_ref[...], preferred_element_type=jnp.float32)
        t = jnp.dot(
            u.astype(jnp.bfloat16), w1_ref[...],
            preferred_element_type=jnp.float32,
        )
        h = jnp.maximum(t + b1_ref[...], 0.0).astype(jnp.bfloat16)
        s2_ref[pl.ds(b0, SB), :] = jnp.dot(
            h, w2_ref[...], preferred_element_type=jnp.float32
        ).astype(jnp.bfloat16)

    @pl.when(m < 2 * NSB)
    def _():
        adjbf_ref[pl.ds(m // 2 * SB, SB),
                  pl.ds(m % 2 * HW, HW)] = adj_ref[...].astype(jnp.bfloat16)


def kernel(x, adj, W1, b1, W2, b2):
    xb = x.astype(jnp.bfloat16)
    w1b = W1.astype(jnp.bfloat16)
    w2b = W2.astype(jnp.bfloat16)
    b1r = b1.reshape(1, D)
    b2r = b2.reshape(1, D)
    return pl.pallas_call(
        _fused_gcn_kernel,
        grid=(2 * NSB + 2,),
        in_specs=[
            pl.BlockSpec(
                (SB, HW),
                lambda i: (jnp.minimum(i, 2 * NSB - 1) // 2,
                           jnp.minimum(i, 2 * NSB - 1) % 2),
            ),
            pl.BlockSpec((N, D), lambda i: (0, 0)),
            pl.BlockSpec((D, D), lambda i: (0, 0)),
            pl.BlockSpec((1, D), lambda i: (0, 0)),
            pl.BlockSpec((D, D), lambda i: (0, 0)),
            pl.BlockSpec((1, D), lambda i: (0, 0)),
        ],
        out_specs=pl.BlockSpec((N, D), lambda i: (0, 0)),
        out_shape=jax.ShapeDtypeStruct((N, D), jnp.float32),
        scratch_shapes=[
            pltpu.VMEM((N, N), jnp.bfloat16),
            pltpu.VMEM((N, D), jnp.bfloat16),
        ],
    )(adj, xb, w1b, b1r, w2b, b2r)
